# Initial kernel scaffold; baseline (speedup 1.0000x reference)
import jax
import jax.numpy as jnp
from jax.experimental import pallas as pl


def kernel(rpe_bias):
    raise NotImplementedError("write your pallas kernel here")


# speedup vs baseline: 24.2736x; 24.2736x over previous
import functools
import jax
import jax.numpy as jnp
from jax import lax
from jax.experimental import pallas as pl
from jax.experimental.pallas import tpu as pltpu
from jax.experimental.pallas import tpu_sc as plsc

_TBL = 3969
_TBLP = 4096
_OFF = 1984


def _rpe_sc_kernel(wt_hbm, out_hbm, w_v, buf_v):
    core = lax.axis_index("c")
    sub = lax.axis_index("s")
    h = sub
    half = core

    pltpu.sync_copy(wt_hbm.at[h], w_v)

    lanes = lax.iota(jnp.int32, 16)

    def slab(g, carry):
        ri = 16 * half + g
        base_ri = _OFF - 63 * ri

        def row(ci, c2):
            base_i = base_ri - ci
            for c in range(64):
                b = base_i + 63 * (c >> 1) + 16 * (c & 1)
                buf_v[ci, pl.ds(16 * c, 16)] = w_v[pl.ds(b, 16)]
            return c2

        lax.fori_loop(0, 32, row, 0)
        pltpu.sync_copy(buf_v, out_hbm.at[h, pl.ds(ri * 32, 32)])
        return carry

    lax.fori_loop(0, 16, slab, 0)


@jax.jit
def kernel(rpe_bias):
    wt = jnp.flip(rpe_bias, axis=0).T
    wt = jnp.pad(wt, ((0, 0), (0, _TBLP - _TBL)))
    mesh = plsc.VectorSubcoreMesh(core_axis_name="c", subcore_axis_name="s")
    run = functools.partial(
        pl.kernel,
        mesh=mesh,
        out_type=jax.ShapeDtypeStruct((16, 1024, 1024), jnp.float32),
        scratch_types=[
            pltpu.VMEM((_TBLP,), jnp.float32),
            pltpu.VMEM((32, 1024), jnp.float32),
        ],
    )(_rpe_sc_kernel)
    return run(wt)


# speedup vs baseline: 69.0183x; 2.8433x over previous
import functools
import jax
import jax.numpy as jnp
from jax import lax
from jax.experimental import pallas as pl
from jax.experimental.pallas import tpu as pltpu
from jax.experimental.pallas import tpu_sc as plsc

_TBL = 3969
_TBLP = 4096
_OFF = 1984


def _rpe_sc_kernel(wt_hbm, out_hbm, w_v, buf_v, sem0, sem1):
    core = lax.axis_index("c")
    sub = lax.axis_index("s")
    h = sub
    half = core
    sems = (sem0, sem1)

    pltpu.sync_copy(wt_hbm.at[h], w_v)

    def fill(g, b):
        ri = 16 * half + g
        base_ri = _OFF - 63 * ri

        @plsc.parallel_loop(0, 32, 1, unroll=2)
        def row(ci):
            base_i = base_ri - ci
            for c in range(64):
                off = 63 * (c >> 1) + 16 * (c & 1)
                buf_v[b, ci, pl.ds(16 * c, 16)] = w_v[pl.ds(base_i + off, 16)]

        return ri

    for b in range(2):
        ri = fill(b, b)
        pltpu.async_copy(buf_v.at[b], out_hbm.at[h, pl.ds(ri * 32, 32)], sems[b])

    def body(g2, carry):
        g = 2 * g2
        for b in range(2):
            pltpu.make_async_copy(
                buf_v.at[b], out_hbm.at[h, pl.ds(0, 32)], sems[b]
            ).wait()
            ri = fill(g + b, b)
            pltpu.async_copy(buf_v.at[b], out_hbm.at[h, pl.ds(ri * 32, 32)], sems[b])
        return carry

    lax.fori_loop(1, 8, body, 0)

    for b in range(2):
        pltpu.make_async_copy(
            buf_v.at[b], out_hbm.at[h, pl.ds(0, 32)], sems[b]
        ).wait()


@jax.jit
def kernel(rpe_bias):
    wt = jnp.flip(rpe_bias, axis=0).T
    wt = jnp.pad(wt, ((0, 0), (0, _TBLP - _TBL)))
    mesh = plsc.VectorSubcoreMesh(core_axis_name="c", subcore_axis_name="s")
    run = functools.partial(
        pl.kernel,
        mesh=mesh,
        out_type=jax.ShapeDtypeStruct((16, 1024, 1024), jnp.float32),
        scratch_types=[
            pltpu.VMEM((_TBLP,), jnp.float32),
            pltpu.VMEM((2, 32, 1024), jnp.float32),
            pltpu.SemaphoreType.DMA,
            pltpu.SemaphoreType.DMA,
        ],
    )(_rpe_sc_kernel)
    return run(wt)
